# Initial kernel scaffold; baseline (speedup 1.0000x reference)
import jax
import jax.numpy as jnp
from jax.experimental import pallas as pl


def kernel(x, adj, W1, a1, W2, a2):
    raise NotImplementedError("write your pallas kernel here")


# speedup vs baseline: 1.3152x; 1.3152x over previous
import functools

import jax
import jax.numpy as jnp
from jax.experimental import pallas as pl
from jax.experimental.pallas import tpu as pltpu

ALPHA = 0.2


def _lrelu(v):
    return jnp.where(v > 0, v, ALPHA * v)


def _elu(v):
    return jnp.where(v > 0, v, jnp.exp(v) - 1.0)


def _proj_body(h_ref, w_ref, a_ref, wh_ref, wh12_ref, mx_ref):
    i = pl.program_id(0)
    wh = jnp.dot(h_ref[...], w_ref[...], preferred_element_type=jnp.float32)
    wh_ref[...] = wh
    wh12 = jnp.dot(wh, a_ref[...], preferred_element_type=jnp.float32)
    wh12_ref[...] = wh12
    nh = wh12.shape[1] // 2
    bm = jnp.max(wh12[:, nh:], axis=0, keepdims=True)

    @pl.when(i == 0)
    def _():
        mx_ref[...] = jnp.full(mx_ref.shape, -jnp.inf, mx_ref.dtype)

    mx_ref[0:1, 0:nh] = jnp.maximum(mx_ref[0:1, 0:nh], bm)


def _project(h, w, a, block_rows):
    n, f = h.shape
    fp = w.shape[1]
    nh2 = a.shape[1]
    grid = (n // block_rows,)
    return pl.pallas_call(
        _proj_body,
        grid=grid,
        in_specs=[
            pl.BlockSpec((block_rows, f), lambda i: (i, 0)),
            pl.BlockSpec((f, fp), lambda i: (0, 0)),
            pl.BlockSpec((fp, nh2), lambda i: (0, 0)),
        ],
        out_specs=[
            pl.BlockSpec((block_rows, fp), lambda i: (i, 0)),
            pl.BlockSpec((block_rows, nh2), lambda i: (i, 0)),
            pl.BlockSpec((8, 128), lambda i: (0, 0)),
        ],
        out_shape=[
            jax.ShapeDtypeStruct((n, fp), jnp.float32),
            jax.ShapeDtypeStruct((n, nh2), jnp.float32),
            jax.ShapeDtypeStruct((8, 128), jnp.float32),
        ],
        compiler_params=pltpu.CompilerParams(
            dimension_semantics=("arbitrary",),
        ),
    )(h, w, a)


def _attn_body(wh12i_ref, wh12t_ref, wh_ref, mx_ref, adj_ref, out_ref,
               num_acc, den_acc, *, nheads, fp, bc, mode):
    j = pl.program_id(1)
    nj = pl.num_programs(1)

    @pl.when(j == 0)
    def _():
        num_acc[...] = jnp.zeros(num_acc.shape, num_acc.dtype)
        den_acc[...] = jnp.zeros(den_acc.shape, den_acc.dtype)

    adj = adj_ref[...]
    col0 = j * bc
    for h in range(nheads):
        wh1 = wh12i_ref[:, h:h + 1]
        wh2 = wh12t_ref[nheads + h:nheads + h + 1, pl.ds(col0, bc)]
        mx = mx_ref[0:1, h:h + 1]
        m = _lrelu(wh1 + mx)
        e = _lrelu(wh1 + wh2)
        p = jnp.where(adj > 0, jnp.exp(e - m), 0.0)
        den_acc[:, h:h + 1] += jnp.sum(p, axis=1, keepdims=True)
        whb = wh_ref[pl.ds(col0, bc), h * fp:(h + 1) * fp]
        num_acc[:, h * fp:(h + 1) * fp] += jnp.dot(
            p, whb, preferred_element_type=jnp.float32)

    @pl.when(j == nj - 1)
    def _():
        if mode == "concat_elu":
            for h in range(nheads):
                v = num_acc[:, h * fp:(h + 1) * fp] / den_acc[:, h:h + 1]
                out_ref[:, h * fp:(h + 1) * fp] = _elu(_elu(v))
        else:
            v = num_acc[...] / den_acc[:, 0:1]
            vmax = jnp.max(v, axis=1, keepdims=True)
            vs = v - vmax
            lse = jnp.log(jnp.sum(jnp.exp(vs), axis=1, keepdims=True))
            out_ref[...] = vs - lse


def _attention(wh12, wh, mx, adj, nheads, fp, br, bc, mode):
    n = adj.shape[0]
    grid = (n // br, n // bc)
    wh12t = wh12.T
    body = functools.partial(_attn_body, nheads=nheads, fp=fp, bc=bc, mode=mode)
    return pl.pallas_call(
        body,
        grid=grid,
        in_specs=[
            pl.BlockSpec((br, 2 * nheads), lambda i, j: (i, 0)),
            pl.BlockSpec((2 * nheads, n), lambda i, j: (0, 0)),
            pl.BlockSpec((n, nheads * fp), lambda i, j: (0, 0)),
            pl.BlockSpec((8, 128), lambda i, j: (0, 0)),
            pl.BlockSpec((br, bc), lambda i, j: (i, j)),
        ],
        out_specs=pl.BlockSpec((br, nheads * fp), lambda i, j: (i, 0)),
        out_shape=jax.ShapeDtypeStruct((n, nheads * fp), jnp.float32),
        scratch_shapes=[
            pltpu.VMEM((br, nheads * fp), jnp.float32),
            pltpu.VMEM((br, 128), jnp.float32),
        ],
        compiler_params=pltpu.CompilerParams(
            dimension_semantics=("arbitrary", "arbitrary"),
        ),
    )(wh12, wh12t, wh, mx, adj)


def kernel(x, adj, W1, a1, W2, a2):
    n, nfeat = x.shape
    nheads, _, nhid = W1.shape
    nclass = W2.shape[1]

    w1cat = jnp.transpose(W1, (1, 0, 2)).reshape(nfeat, nheads * nhid)
    a1m = jnp.zeros((nheads * nhid, 2 * nheads), dtype=jnp.float32)
    for h in range(nheads):
        a1m = a1m.at[h * nhid:(h + 1) * nhid, h].set(a1[h, :nhid, 0])
        a1m = a1m.at[h * nhid:(h + 1) * nhid, nheads + h].set(a1[h, nhid:, 0])
    a2m = jnp.concatenate([a2[:nclass], a2[nclass:]], axis=1)

    br = min(256, n)
    bc = min(1024, n)

    wh, wh12, mx = _project(x, w1cat, a1m, br)
    h1 = _attention(wh12, wh, mx, adj, nheads, nhid, br, bc, "concat_elu")

    whp, wh12p, mxp = _project(h1, W2, a2m, br)
    out = _attention(wh12p, whp, mxp, adj, 1, nclass, br, bc, "log_softmax")
    return out


# speedup vs baseline: 1.4203x; 1.0799x over previous
import functools

import jax
import jax.numpy as jnp
from jax.experimental import pallas as pl
from jax.experimental.pallas import tpu as pltpu

ALPHA = 0.2


def _lrelu(v):
    return jnp.where(v > 0, v, ALPHA * v)


def _elu(v):
    return jnp.where(v > 0, v, jnp.exp(v) - 1.0)


def _proj_body(h_ref, w_ref, a_ref, wh_ref, wh12_ref, mx_ref):
    i = pl.program_id(0)
    wh = jnp.dot(h_ref[...], w_ref[...], preferred_element_type=jnp.float32)
    wh_ref[...] = wh
    wh12 = jnp.dot(wh, a_ref[...], preferred_element_type=jnp.float32)
    wh12_ref[...] = wh12
    nh = wh12.shape[1] // 2
    bm = jnp.max(wh12[:, nh:], axis=0, keepdims=True)

    @pl.when(i == 0)
    def _():
        mx_ref[...] = jnp.full(mx_ref.shape, -jnp.inf, mx_ref.dtype)

    mx_ref[0:1, 0:nh] = jnp.maximum(mx_ref[0:1, 0:nh], bm)


def _project(h, w, a, block_rows):
    n, f = h.shape
    fp = w.shape[1]
    nh2 = a.shape[1]
    grid = (n // block_rows,)
    return pl.pallas_call(
        _proj_body,
        grid=grid,
        in_specs=[
            pl.BlockSpec((block_rows, f), lambda i: (i, 0)),
            pl.BlockSpec((f, fp), lambda i: (0, 0)),
            pl.BlockSpec((fp, nh2), lambda i: (0, 0)),
        ],
        out_specs=[
            pl.BlockSpec((block_rows, fp), lambda i: (i, 0)),
            pl.BlockSpec((block_rows, nh2), lambda i: (i, 0)),
            pl.BlockSpec((8, 128), lambda i: (0, 0)),
        ],
        out_shape=[
            jax.ShapeDtypeStruct((n, fp), jnp.float32),
            jax.ShapeDtypeStruct((n, nh2), jnp.float32),
            jax.ShapeDtypeStruct((8, 128), jnp.float32),
        ],
        compiler_params=pltpu.CompilerParams(
            dimension_semantics=("arbitrary",),
        ),
    )(h, w, a)


def _attn_body(wh12i_ref, wh12t_ref, wh_ref, mx_ref, adj_ref, out_ref,
               num_acc, den_acc, *, nheads, fp, bc, mode):
    j = pl.program_id(1)
    nj = pl.num_programs(1)

    @pl.when(j == 0)
    def _():
        num_acc[...] = jnp.zeros(num_acc.shape, num_acc.dtype)
        den_acc[...] = jnp.zeros(den_acc.shape, den_acc.dtype)

    adj = adj_ref[...]
    col0 = j * bc
    for h in range(nheads):
        wh1 = wh12i_ref[:, h:h + 1]
        wh2 = wh12t_ref[nheads + h:nheads + h + 1, pl.ds(col0, bc)]
        mx = mx_ref[0:1, h:h + 1]
        m = _lrelu(wh1 + mx)
        u = jnp.exp(wh1 - m)
        u2 = jnp.exp(ALPHA * wh1 - m)
        v = jnp.exp(wh2)
        v2 = jnp.exp(ALPHA * wh2)
        cond = wh2 > -wh1
        p = jnp.where(cond, u * v, u2 * v2) * adj
        den_acc[:, h:h + 1] += jnp.sum(p, axis=1, keepdims=True)
        whb = wh_ref[pl.ds(col0, bc), h * fp:(h + 1) * fp]
        num_acc[:, h * fp:(h + 1) * fp] += jnp.dot(
            p, whb, preferred_element_type=jnp.float32)

    @pl.when(j == nj - 1)
    def _():
        if mode == "concat_elu":
            for h in range(nheads):
                v = num_acc[:, h * fp:(h + 1) * fp] / den_acc[:, h:h + 1]
                out_ref[:, h * fp:(h + 1) * fp] = _elu(_elu(v))
        else:
            v = num_acc[...] / den_acc[:, 0:1]
            vmax = jnp.max(v, axis=1, keepdims=True)
            vs = v - vmax
            lse = jnp.log(jnp.sum(jnp.exp(vs), axis=1, keepdims=True))
            out_ref[...] = vs - lse


def _attention(wh12, wh, mx, adj, nheads, fp, br, bc, mode):
    n = adj.shape[0]
    grid = (n // br, n // bc)
    wh12t = wh12.T
    body = functools.partial(_attn_body, nheads=nheads, fp=fp, bc=bc, mode=mode)
    return pl.pallas_call(
        body,
        grid=grid,
        in_specs=[
            pl.BlockSpec((br, 2 * nheads), lambda i, j: (i, 0)),
            pl.BlockSpec((2 * nheads, n), lambda i, j: (0, 0)),
            pl.BlockSpec((n, nheads * fp), lambda i, j: (0, 0)),
            pl.BlockSpec((8, 128), lambda i, j: (0, 0)),
            pl.BlockSpec((br, bc), lambda i, j: (i, j)),
        ],
        out_specs=pl.BlockSpec((br, nheads * fp), lambda i, j: (i, 0)),
        out_shape=jax.ShapeDtypeStruct((n, nheads * fp), jnp.float32),
        scratch_shapes=[
            pltpu.VMEM((br, nheads * fp), jnp.float32),
            pltpu.VMEM((br, 128), jnp.float32),
        ],
        compiler_params=pltpu.CompilerParams(
            dimension_semantics=("arbitrary", "arbitrary"),
        ),
    )(wh12, wh12t, wh, mx, adj)


def kernel(x, adj, W1, a1, W2, a2):
    n, nfeat = x.shape
    nheads, _, nhid = W1.shape
    nclass = W2.shape[1]

    w1cat = jnp.transpose(W1, (1, 0, 2)).reshape(nfeat, nheads * nhid)
    a1m = jnp.zeros((nheads * nhid, 2 * nheads), dtype=jnp.float32)
    for h in range(nheads):
        a1m = a1m.at[h * nhid:(h + 1) * nhid, h].set(a1[h, :nhid, 0])
        a1m = a1m.at[h * nhid:(h + 1) * nhid, nheads + h].set(a1[h, nhid:, 0])
    a2m = jnp.concatenate([a2[:nclass], a2[nclass:]], axis=1)

    br = min(256, n)
    bc = min(1024, n)

    wh, wh12, mx = _project(x, w1cat, a1m, br)
    h1 = _attention(wh12, wh, mx, adj, nheads, nhid, br, bc, "concat_elu")

    whp, wh12p, mxp = _project(h1, W2, a2m, br)
    out = _attention(wh12p, whp, mxp, adj, 1, nclass, br, bc, "log_softmax")
    return out
